# baseline (speedup 1.0000x reference)
import functools

import jax
import jax.numpy as jnp
from jax.experimental import pallas as pl
from jax.experimental.pallas import tpu as pltpu


def _se_kernel(x_ref, w1_ref, b1_ref, w2_ref, b2_ref, o_ref, *, inv_hw):
    x = x_ref[...]

    s = jnp.sum(x, axis=-1, keepdims=True) * inv_hw
    h = jax.lax.dot_general(
        s, w1_ref[...], (((1,), (1,)), ((), ())),
        preferred_element_type=jnp.float32)
    h = jnp.maximum(h + b1_ref[...].T[None], 0.0)
    g = jax.lax.dot_general(
        h, w2_ref[...], (((2,), (1,)), ((), ())),
        preferred_element_type=jnp.float32)
    g = jax.nn.sigmoid(g + b2_ref[...].T[None])

    o_ref[...] = x * jnp.swapaxes(g, 1, 2)


@jax.jit
def _se_forward(x_nchw, w1, b1, w2, b2):
    n, c, h, w = x_nchw.shape
    cmid = w1.shape[0]
    hw = h * w

    x3 = x_nchw.reshape(n, c, hw)
    b1r = b1.reshape(cmid, 1)
    b2r = b2.reshape(c, 1)

    nb = 32
    while nb > 1 and n % nb:
        nb //= 2
    grid = (n // nb,)

    out3 = pl.pallas_call(
        functools.partial(_se_kernel, inv_hw=1.0 / hw),
        out_shape=jax.ShapeDtypeStruct((n, c, hw), x3.dtype),
        grid_spec=pl.GridSpec(
            grid=grid,
            in_specs=[
                pl.BlockSpec((nb, c, hw), lambda i: (i, 0, 0)),
                pl.BlockSpec((cmid, c), lambda i: (0, 0)),
                pl.BlockSpec((cmid, 1), lambda i: (0, 0)),
                pl.BlockSpec((c, cmid), lambda i: (0, 0)),
                pl.BlockSpec((c, 1), lambda i: (0, 0)),
            ],
            out_specs=pl.BlockSpec((nb, c, hw), lambda i: (i, 0, 0)),
        ),
        compiler_params=pltpu.CompilerParams(
            dimension_semantics=("parallel",),
            vmem_limit_bytes=60 << 20,
        ),
    )(x3, w1, b1r, w2, b2r)
    return out3.reshape(n, c, h, w)


def kernel(x_nchw, w1, b1, w2, b2):
    return _se_forward(x_nchw, w1, b1, w2, b2)


# speedup vs baseline: 1.0200x; 1.0200x over previous
import functools

import jax
import jax.numpy as jnp
from jax.experimental import pallas as pl
from jax.experimental.pallas import tpu as pltpu


def _se_kernel(x_ref, w1ts_ref, b1_ref, w2t_ref, b2_ref, o_ref):
    x = x_ref[...]

    s = jnp.sum(x, axis=-1)
    h = jnp.dot(s, w1ts_ref[...], preferred_element_type=jnp.float32)
    h = jnp.maximum(h + b1_ref[...], 0.0)
    g = jnp.dot(h, w2t_ref[...], preferred_element_type=jnp.float32)
    g = jax.nn.sigmoid(g + b2_ref[...])

    o_ref[...] = x * g[:, :, None]


@jax.jit
def _se_forward(x_nchw, w1, b1, w2, b2):
    n, c, h, w = x_nchw.shape
    cmid = w1.shape[0]
    hw = h * w

    x3 = x_nchw.reshape(n, c, hw)
    w1ts = w1.T * (1.0 / hw)
    w2t = w2.T
    b1r = b1.reshape(1, cmid)
    b2r = b2.reshape(1, c)

    nb = 32
    while nb > 1 and n % nb:
        nb //= 2
    grid = (n // nb,)

    out3 = pl.pallas_call(
        _se_kernel,
        out_shape=jax.ShapeDtypeStruct((n, c, hw), x3.dtype),
        grid_spec=pl.GridSpec(
            grid=grid,
            in_specs=[
                pl.BlockSpec((nb, c, hw), lambda i: (i, 0, 0)),
                pl.BlockSpec((c, cmid), lambda i: (0, 0)),
                pl.BlockSpec((1, cmid), lambda i: (0, 0)),
                pl.BlockSpec((cmid, c), lambda i: (0, 0)),
                pl.BlockSpec((1, c), lambda i: (0, 0)),
            ],
            out_specs=pl.BlockSpec((nb, c, hw), lambda i: (i, 0, 0)),
        ),
        compiler_params=pltpu.CompilerParams(
            dimension_semantics=("parallel",),
            vmem_limit_bytes=60 << 20,
        ),
    )(x3, w1ts, b1r, w2t, b2r)
    return out3.reshape(n, c, h, w)


def kernel(x_nchw, w1, b1, w2, b2):
    return _se_forward(x_nchw, w1, b1, w2, b2)


# speedup vs baseline: 1.0228x; 1.0027x over previous
import functools

import jax
import jax.numpy as jnp
from jax.experimental import pallas as pl
from jax.experimental.pallas import tpu as pltpu


def _se_kernel(x_ref, w1ts_ref, b1_ref, w2t_ref, b2_ref, o_ref):
    x = x_ref[...]

    s = jnp.sum(x, axis=-1)
    h = jnp.dot(s, w1ts_ref[...], preferred_element_type=jnp.float32)
    h = jnp.maximum(h + b1_ref[...], 0.0)
    g = jnp.dot(h, w2t_ref[...], preferred_element_type=jnp.float32)
    g = jax.nn.sigmoid(g + b2_ref[...])

    o_ref[...] = x * g[:, :, None]


@jax.jit
def _se_forward(x_nchw, w1, b1, w2, b2):
    n, c, h, w = x_nchw.shape
    cmid = w1.shape[0]
    hw = h * w

    x3 = x_nchw.reshape(n, c, hw)
    w1ts = w1.T * (1.0 / hw)
    w2t = w2.T
    b1r = b1.reshape(1, cmid)
    b2r = b2.reshape(1, c)

    nb = 32
    while nb > 1 and n % nb:
        nb //= 2
    grid = (n // nb,)

    out3 = pl.pallas_call(
        _se_kernel,
        out_shape=jax.ShapeDtypeStruct((n, c, hw), x3.dtype),
        grid_spec=pl.GridSpec(
            grid=grid,
            in_specs=[
                pl.BlockSpec((nb, c, hw), lambda i: (i, 0, 0)),
                pl.BlockSpec((c, cmid), lambda i: (0, 0)),
                pl.BlockSpec((1, cmid), lambda i: (0, 0)),
                pl.BlockSpec((cmid, c), lambda i: (0, 0)),
                pl.BlockSpec((1, c), lambda i: (0, 0)),
            ],
            out_specs=pl.BlockSpec((nb, c, hw), lambda i: (i, 0, 0)),
        ),
        compiler_params=pltpu.CompilerParams(
            dimension_semantics=("arbitrary",),
            vmem_limit_bytes=60 << 20,
        ),
    )(x3, w1ts, b1r, w2t, b2r)
    return out3.reshape(n, c, h, w)


def kernel(x_nchw, w1, b1, w2, b2):
    return _se_forward(x_nchw, w1, b1, w2, b2)


# speedup vs baseline: 1.0236x; 1.0008x over previous
import functools

import jax
import jax.numpy as jnp
from jax.experimental import pallas as pl
from jax.experimental.pallas import tpu as pltpu


def _se_kernel(x_ref, w1ts_ref, b1_ref, w2t_ref, b2_ref, o_ref):
    x = x_ref[...]

    s = jnp.sum(x, axis=-1)
    h = jnp.dot(s, w1ts_ref[...], preferred_element_type=jnp.float32)
    h = jnp.maximum(h + b1_ref[...], 0.0)
    g = jnp.dot(h, w2t_ref[...], preferred_element_type=jnp.float32)
    g = jax.nn.sigmoid(g + b2_ref[...])

    o_ref[...] = x * g[:, :, None]


@jax.jit
def _se_forward(x_nchw, w1, b1, w2, b2):
    n, c, h, w = x_nchw.shape
    cmid = w1.shape[0]
    hw = h * w

    x3 = x_nchw.reshape(n, c, hw)
    w1ts = w1.T * (1.0 / hw)
    w2t = w2.T
    b1r = b1.reshape(1, cmid)
    b2r = b2.reshape(1, c)

    nb = 32
    while nb > 1 and n % nb:
        nb //= 2
    grid = (n // nb,)

    out3 = pl.pallas_call(
        _se_kernel,
        out_shape=jax.ShapeDtypeStruct((n, c, hw), x3.dtype),
        grid_spec=pl.GridSpec(
            grid=grid,
            in_specs=[
                pl.BlockSpec((nb, c, hw), lambda i: (i, 0, 0)),
                pl.BlockSpec((c, cmid), lambda i: (0, 0)),
                pl.BlockSpec((1, cmid), lambda i: (0, 0)),
                pl.BlockSpec((cmid, c), lambda i: (0, 0)),
                pl.BlockSpec((1, c), lambda i: (0, 0)),
            ],
            out_specs=pl.BlockSpec((nb, c, hw), lambda i: (i, 0, 0)),
        ),
        compiler_params=pltpu.CompilerParams(
            dimension_semantics=("parallel",),
            vmem_limit_bytes=60 << 20,
        ),
    )(x3, w1ts, b1r, w2t, b2r)
    return out3.reshape(n, c, h, w)


def kernel(x_nchw, w1, b1, w2, b2):
    return _se_forward(x_nchw, w1, b1, w2, b2)
